# baseline (speedup 1.0000x reference)
import functools

import jax
import jax.numpy as jnp
import numpy as np
from jax import lax
from jax.experimental import pallas as pl
from jax.experimental.pallas import tpu as pltpu
from jax.experimental.pallas import tpu_sc as plsc

_IU = 256
_BU = 64
_N = 128
_N1 = _N + 1
_NP = 256
_OU = 64
_B = 8
_T = 512

_pallas_call = pl.pallas_call


def _attn_body(x_ref, wq_ref, bq_ref, wk_ref, bk_ref, wa_ref, ba_ref,
               we_ref, be_ref, wk_out, al_out, e_out):
    x = x_ref[0]
    q = jnp.dot(x, wq_ref[...]) + bq_ref[...]
    k = jnp.dot(x, wk_ref[...]) + bk_ref[...]
    am = jnp.dot(x, wa_ref[...]) + ba_ref[...]
    e = jnp.maximum(jnp.dot(x, we_ref[...]) + be_ref[...], 0.0)
    s1 = jax.lax.dot_general(q, k, (((1,), (1,)), ((), ()))) / np.power(
        _N1, 0.5)
    s2 = jnp.dot(s1, am)
    m = jnp.max(s2, axis=0, keepdims=True)
    ez = jnp.exp(s2 - m)
    zs = jnp.sum(ez, axis=0, keepdims=True)
    sm = ez / zs
    col = jax.lax.broadcasted_iota(jnp.int32, (_T, _NP), 1)
    smx = jnp.where(col < _N1, sm, jnp.inf)
    minv = jnp.min(smx, axis=1, keepdims=True)
    idx = jnp.min(jnp.where(smx == minv, col, jnp.int32(1 << 30)), axis=1)
    wk_out[0, 0] = idx
    al_out[0, 0] = sm[_T - 1, :]
    e_out[0] = e


_NEV = _T * _B


_L = 16


def _sc_prev_body(wk_hbm, p_hbm, gl_hbm, wk_v, p_v, gl_v, last_sm):
    cid = lax.axis_index("c")
    sid = lax.axis_index("s")

    @pl.when((cid == 0) & (sid == 0))
    def _():
        pltpu.sync_copy(wk_hbm, wk_v)

        def li(n, _):
            last_sm[n] = -1
            return 0

        lax.fori_loop(0, _N, li, 0)
        lanes = lax.iota(jnp.int32, _L)

        def outer(g, _):
            vec = jnp.zeros((_L,), jnp.int32)
            wkvec = wk_v[pl.ds(g * _L, _L)]
            for r in range(_L):
                i = g * _L + r
                n = wkvec[r]
                valid = n < _N
                sn = jnp.minimum(n, _N - 1)
                pv = jnp.where(valid, last_sm[sn], -1)
                vec = jnp.where(lanes == r, pv, vec)

                @pl.when(valid)
                def _w():
                    last_sm[sn] = i
            p_v[pl.ds(g * _L, _L)] = vec
            return 0

        lax.fori_loop(0, _NEV // _L, outer, 0)

        def gout(g, _):
            vec = jnp.zeros((_L,), jnp.int32)
            for r in range(_L):
                vec = jnp.where(lanes == r, last_sm[g * _L + r], vec)
            gl_v[pl.ds(g * _L, _L)] = vec
            return 0

        lax.fori_loop(0, _N // _L, gout, 0)
        pltpu.sync_copy(p_v, p_hbm)
        pltpu.sync_copy(gl_v, gl_hbm)


@functools.lru_cache(maxsize=1)
def _sc_prev_call():
    mesh = plsc.VectorSubcoreMesh(core_axis_name="c", subcore_axis_name="s")
    return functools.partial(
        pl.kernel,
        mesh=mesh,
        out_type=[
            jax.ShapeDtypeStruct((_NEV,), jnp.int32),
            jax.ShapeDtypeStruct((_N,), jnp.int32),
        ],
        scratch_types=[
            pltpu.VMEM((_NEV,), jnp.int32),
            pltpu.VMEM((_NEV,), jnp.int32),
            pltpu.VMEM((_N,), jnp.int32),
            pltpu.SMEM((_N,), jnp.int32),
        ],
    )(_sc_prev_body)


def _scan_body(wkt_ref, p_ref, gl_ref, atv_ref, e_ref, wl_ref, bl_ref,
               out_ref, mf_ref, d_ref, s3_ref):
    hi = jax.lax.Precision.HIGHEST
    d_ref[pl.ds(_NEV, 1), :] = jnp.zeros((1, _OU), jnp.float32)

    def p13(t, _):
        dns = []
        dps = []
        crows = []
        for j in range(_B):
            n = wkt_ref[t, j]
            valid = n < _N
            sn = jnp.minimum(n, _N - 1)
            i = t * _B + j
            e = e_ref[pl.ds(j * _T + t, 1), :]
            wb = wl_ref[pl.ds(sn * _BU, _BU), :]
            dnew = jnp.dot(e, wb, preferred_element_type=jnp.float32,
                           precision=hi)
            d_ref[pl.ds(i, 1), :] = dnew
            dns.append(dnew)
            prev = p_ref[i]
            sp = jnp.where(prev < 0, _NEV, prev)
            dps.append(d_ref[pl.ds(sp, 1), :])
            arow = atv_ref[pl.ds(sn, 1), :]
            crows.append(jnp.where(valid, arow, 0.0))
        delta = jnp.concatenate(dns, axis=0) - jnp.concatenate(dps, axis=0)
        c = jnp.concatenate(crows, axis=0)
        s = jax.lax.dot_general(c, delta, (((0,), (0,)), ((), ())),
                                precision=hi)
        s3_ref[pl.ds(t, 1)] = jnp.reshape(s, (1, _B, _OU))
        return 0

    jax.lax.fori_loop(0, _T, p13, 0)

    row = jax.lax.broadcasted_iota(jnp.int32, (_T, _T), 0)
    colt = jax.lax.broadcasted_iota(jnp.int32, (_T, _T), 1)
    ltri = jnp.where(row > colt, 1.0, 0.0).astype(jnp.float32)
    out1 = jax.lax.dot_general(ltri, s3_ref[...], (((1,), (0,)), ((), ())),
                               precision=hi)
    out_ref[...] = out1 + jnp.reshape(bl_ref[...], (1, 1, _OU))
    wl_last = wl_ref[pl.ds(_N * _BU, _BU), :]
    for b in range(_B):
        eb = e_ref[pl.ds(b * _T, _T), :]
        o2 = jnp.dot(eb, wl_last, preferred_element_type=jnp.float32,
                     precision=hi) * atv_ref[pl.ds(_N, 1), pl.ds(b, 1)]
        out_ref[:, pl.ds(b, 1), :] += o2[:, None, :]

    def p5(n, _):
        li = gl_ref[n]
        t = li // _B
        j = li - t * _B
        erow = e_ref[pl.ds(j * _T + jnp.maximum(t, 0), 1), :]
        mf_ref[pl.ds(n, 1), :] = jnp.where(li >= 0, erow, 0.0)
        return 0

    jax.lax.fori_loop(0, _N, p5, 0)


def kernel(inp, Wq, bq, Wk, bk, Wa, ba, We, be, Wl, bl):
    f32 = jnp.float32
    pad = _NP - _N1
    wq_p = jnp.pad(Wq, ((0, 0), (0, pad)))
    wk_p = jnp.pad(Wk, ((0, 0), (0, pad)))
    wa_p = jnp.pad(Wa, ((0, 0), (0, pad)))
    bq_p = jnp.pad(bq, (0, pad)).reshape(1, _NP)
    bk_p = jnp.pad(bk, (0, pad)).reshape(1, _NP)
    ba_p = jnp.pad(ba, (0, pad)).reshape(1, _NP)
    be2 = be.reshape(1, _BU)

    wfull = pl.BlockSpec((_IU, _NP), lambda b: (0, 0))
    bfull = pl.BlockSpec((1, _NP), lambda b: (0, 0))
    weakest, al, e_all = _pallas_call(
        _attn_body,
        grid=(_B,),
        in_specs=[
            pl.BlockSpec((1, _T, _IU), lambda b: (b, 0, 0)),
            wfull, bfull, wfull, bfull, wfull, bfull,
            pl.BlockSpec((_IU, _BU), lambda b: (0, 0)),
            pl.BlockSpec((1, _BU), lambda b: (0, 0)),
        ],
        out_specs=[
            pl.BlockSpec((1, 1, _T), lambda b: (b, 0, 0)),
            pl.BlockSpec((1, 1, _NP), lambda b: (b, 0, 0)),
            pl.BlockSpec((1, _T, _BU), lambda b: (b, 0, 0)),
        ],
        out_shape=[
            jax.ShapeDtypeStruct((_B, 1, _T), jnp.int32),
            jax.ShapeDtypeStruct((_B, 1, _NP), f32),
            jax.ShapeDtypeStruct((_B, _T, _BU), f32),
        ],
    )(inp, wq_p, bq_p, wk_p, bk_p, wa_p, ba_p, We, be2)

    wkt = weakest[:, 0, :].T
    at = al[:, 0, :_N1].T
    e_flat = e_all.reshape(_B * _T, _BU)
    bl2 = bl.reshape(1, _OU)
    p_evt, gl = _sc_prev_call()(wkt.reshape(_NEV))

    out_pre, mfin = _pallas_call(
        _scan_body,
        in_specs=[
            pl.BlockSpec(memory_space=pltpu.SMEM),
            pl.BlockSpec(memory_space=pltpu.SMEM),
            pl.BlockSpec(memory_space=pltpu.SMEM),
            pl.BlockSpec(memory_space=pltpu.VMEM),
            pl.BlockSpec(memory_space=pltpu.VMEM),
            pl.BlockSpec(memory_space=pltpu.VMEM),
            pl.BlockSpec(memory_space=pltpu.VMEM),
        ],
        out_specs=[
            pl.BlockSpec(memory_space=pltpu.VMEM),
            pl.BlockSpec(memory_space=pltpu.VMEM),
        ],
        out_shape=[
            jax.ShapeDtypeStruct((_T, _B, _OU), f32),
            jax.ShapeDtypeStruct((_N, _BU), f32),
        ],
        scratch_shapes=[
            pltpu.VMEM((_NEV + 8, _OU), f32),
            pltpu.VMEM((_T, _B, _OU), f32),
        ],
    )(wkt, p_evt, gl, at, e_flat, Wl, bl2)

    output = out_pre.reshape(_B, _T, _OU)
    final_mem = jnp.broadcast_to(mfin[None], (_B, _N, _BU))
    return output, final_mem


# speedup vs baseline: 1.1108x; 1.1108x over previous
import functools

import jax
import jax.numpy as jnp
import numpy as np
from jax import lax
from jax.experimental import pallas as pl
from jax.experimental.pallas import tpu as pltpu
from jax.experimental.pallas import tpu_sc as plsc

_IU = 256
_BU = 64
_N = 128
_N1 = _N + 1
_NP = 256
_OU = 64
_B = 8
_T = 512

_pallas_call = pl.pallas_call


def _attn_body(x_ref, wq_ref, bq_ref, wk_ref, bk_ref, wa_ref, ba_ref,
               we_ref, be_ref, wk_out, al_out, e_out):
    x = x_ref[0]
    q = jnp.dot(x, wq_ref[...]) + bq_ref[...]
    k = jnp.dot(x, wk_ref[...]) + bk_ref[...]
    am = jnp.dot(x, wa_ref[...]) + ba_ref[...]
    e = jnp.maximum(jnp.dot(x, we_ref[...]) + be_ref[...], 0.0)
    s1 = jax.lax.dot_general(q, k, (((1,), (1,)), ((), ()))) / np.power(
        _N1, 0.5)
    s2 = jnp.dot(s1, am)
    m = jnp.max(s2, axis=0, keepdims=True)
    ez = jnp.exp(s2 - m)
    zs = jnp.sum(ez, axis=0, keepdims=True)
    sm = ez / zs
    col = jax.lax.broadcasted_iota(jnp.int32, (_T, _NP), 1)
    smx = jnp.where(col < _N1, sm, jnp.inf)
    minv = jnp.min(smx, axis=1, keepdims=True)
    idx = jnp.min(jnp.where(smx == minv, col, jnp.int32(1 << 30)), axis=1)
    wk_out[0, 0] = idx
    al_out[0, 0] = sm[_T - 1, :]
    e_out[0] = e


_NEV = _T * _B


_L = 16


def _sc_prev_body(wk_hbm, p_hbm, gl_hbm, wk_v, p_v, gl_v, last_sm):
    cid = lax.axis_index("c")
    sid = lax.axis_index("s")

    @pl.when((cid == 0) & (sid == 0))
    def _():
        pltpu.sync_copy(wk_hbm, wk_v)

        def li(n, _):
            last_sm[n] = -1
            return 0

        lax.fori_loop(0, _N, li, 0)
        lanes = lax.iota(jnp.int32, _L)

        def outer(g, _):
            vec = jnp.zeros((_L,), jnp.int32)
            wkvec = wk_v[pl.ds(g * _L, _L)]
            for r in range(_L):
                i = g * _L + r
                n = wkvec[r]
                valid = n < _N
                sn = jnp.minimum(n, _N - 1)
                pv = jnp.where(valid, last_sm[sn], -1)
                vec = jnp.where(lanes == r, pv, vec)

                @pl.when(valid)
                def _w():
                    last_sm[sn] = i
            p_v[pl.ds(g * _L, _L)] = vec
            return 0

        lax.fori_loop(0, _NEV // _L, outer, 0)

        def gout(g, _):
            vec = jnp.zeros((_L,), jnp.int32)
            for r in range(_L):
                vec = jnp.where(lanes == r, last_sm[g * _L + r], vec)
            gl_v[pl.ds(g * _L, _L)] = vec
            return 0

        lax.fori_loop(0, _N // _L, gout, 0)
        pltpu.sync_copy(p_v, p_hbm)
        pltpu.sync_copy(gl_v, gl_hbm)


@functools.lru_cache(maxsize=1)
def _sc_prev_call():
    mesh = plsc.VectorSubcoreMesh(core_axis_name="c", subcore_axis_name="s")
    return functools.partial(
        pl.kernel,
        mesh=mesh,
        out_type=[
            jax.ShapeDtypeStruct((_NEV,), jnp.int32),
            jax.ShapeDtypeStruct((_N,), jnp.int32),
        ],
        scratch_types=[
            pltpu.VMEM((_NEV,), jnp.int32),
            pltpu.VMEM((_NEV,), jnp.int32),
            pltpu.VMEM((_N,), jnp.int32),
            pltpu.SMEM((_N,), jnp.int32),
        ],
    )(_sc_prev_body)


def _scan_body(wkt_ref, p_ref, gl_ref, atv_ref, e_ref, wl_ref, bl_ref,
               out_ref, mf_ref, d_ref, s3_ref):
    hi = jax.lax.Precision.HIGHEST
    d_ref[pl.ds(_NEV, 1), :] = jnp.zeros((1, _OU), jnp.float32)

    def p13(g, _):
        for u in range(2):
            t = g * 2 + u
            dns = []
            dps = []
            crows = []
            for j in range(_B):
                n = wkt_ref[t, j]
                valid = n < _N
                sn = jnp.minimum(n, _N - 1)
                i = t * _B + j
                e = e_ref[pl.ds(j * _T + t, 1), :]
                wb = wl_ref[pl.ds(sn * _BU, _BU), :]
                dnew = jnp.dot(e, wb, preferred_element_type=jnp.float32,
                               precision=hi)
                d_ref[pl.ds(i, 1), :] = dnew
                dns.append(dnew)
                prev = p_ref[i]
                sp = jnp.where(prev < 0, _NEV, prev)
                dps.append(d_ref[pl.ds(sp, 1), :])
                arow = atv_ref[pl.ds(sn, 1), :]
                crows.append(jnp.where(valid, arow, 0.0))
            delta = (jnp.concatenate(dns, axis=0)
                     - jnp.concatenate(dps, axis=0))
            c = jnp.concatenate(crows, axis=0)
            s = jax.lax.dot_general(c, delta, (((0,), (0,)), ((), ())),
                                    precision=hi)
            s3_ref[pl.ds(t, 1)] = jnp.reshape(s, (1, _B, _OU))
        return 0

    jax.lax.fori_loop(0, _T // 2, p13, 0)

    row = jax.lax.broadcasted_iota(jnp.int32, (_T, _T), 0)
    colt = jax.lax.broadcasted_iota(jnp.int32, (_T, _T), 1)
    ltri = jnp.where(row > colt, 1.0, 0.0).astype(jnp.float32)
    out1 = jax.lax.dot_general(ltri, s3_ref[...], (((1,), (0,)), ((), ())),
                               precision=hi)
    out_ref[...] = out1 + jnp.reshape(bl_ref[...], (1, 1, _OU))
    wl_last = wl_ref[pl.ds(_N * _BU, _BU), :]
    for b in range(_B):
        eb = e_ref[pl.ds(b * _T, _T), :]
        o2 = jnp.dot(eb, wl_last, preferred_element_type=jnp.float32,
                     precision=hi) * atv_ref[pl.ds(_N, 1), pl.ds(b, 1)]
        out_ref[:, pl.ds(b, 1), :] += o2[:, None, :]

    def p5(n, _):
        li = gl_ref[n]
        t = li // _B
        j = li - t * _B
        erow = e_ref[pl.ds(j * _T + jnp.maximum(t, 0), 1), :]
        mf_ref[pl.ds(n, 1), :] = jnp.where(li >= 0, erow, 0.0)
        return 0

    jax.lax.fori_loop(0, _N, p5, 0)


def kernel(inp, Wq, bq, Wk, bk, Wa, ba, We, be, Wl, bl):
    f32 = jnp.float32
    pad = _NP - _N1
    wq_p = jnp.pad(Wq, ((0, 0), (0, pad)))
    wk_p = jnp.pad(Wk, ((0, 0), (0, pad)))
    wa_p = jnp.pad(Wa, ((0, 0), (0, pad)))
    bq_p = jnp.pad(bq, (0, pad)).reshape(1, _NP)
    bk_p = jnp.pad(bk, (0, pad)).reshape(1, _NP)
    ba_p = jnp.pad(ba, (0, pad)).reshape(1, _NP)
    be2 = be.reshape(1, _BU)

    wfull = pl.BlockSpec((_IU, _NP), lambda b: (0, 0))
    bfull = pl.BlockSpec((1, _NP), lambda b: (0, 0))
    weakest, al, e_all = _pallas_call(
        _attn_body,
        grid=(_B,),
        in_specs=[
            pl.BlockSpec((1, _T, _IU), lambda b: (b, 0, 0)),
            wfull, bfull, wfull, bfull, wfull, bfull,
            pl.BlockSpec((_IU, _BU), lambda b: (0, 0)),
            pl.BlockSpec((1, _BU), lambda b: (0, 0)),
        ],
        out_specs=[
            pl.BlockSpec((1, 1, _T), lambda b: (b, 0, 0)),
            pl.BlockSpec((1, 1, _NP), lambda b: (b, 0, 0)),
            pl.BlockSpec((1, _T, _BU), lambda b: (b, 0, 0)),
        ],
        out_shape=[
            jax.ShapeDtypeStruct((_B, 1, _T), jnp.int32),
            jax.ShapeDtypeStruct((_B, 1, _NP), f32),
            jax.ShapeDtypeStruct((_B, _T, _BU), f32),
        ],
    )(inp, wq_p, bq_p, wk_p, bk_p, wa_p, ba_p, We, be2)

    wkt = weakest[:, 0, :].T
    at = al[:, 0, :_N1].T
    e_flat = e_all.reshape(_B * _T, _BU)
    bl2 = bl.reshape(1, _OU)
    p_evt, gl = _sc_prev_call()(wkt.reshape(_NEV))

    out_pre, mfin = _pallas_call(
        _scan_body,
        in_specs=[
            pl.BlockSpec(memory_space=pltpu.SMEM),
            pl.BlockSpec(memory_space=pltpu.SMEM),
            pl.BlockSpec(memory_space=pltpu.SMEM),
            pl.BlockSpec(memory_space=pltpu.VMEM),
            pl.BlockSpec(memory_space=pltpu.VMEM),
            pl.BlockSpec(memory_space=pltpu.VMEM),
            pl.BlockSpec(memory_space=pltpu.VMEM),
        ],
        out_specs=[
            pl.BlockSpec(memory_space=pltpu.VMEM),
            pl.BlockSpec(memory_space=pltpu.VMEM),
        ],
        out_shape=[
            jax.ShapeDtypeStruct((_T, _B, _OU), f32),
            jax.ShapeDtypeStruct((_N, _BU), f32),
        ],
        scratch_shapes=[
            pltpu.VMEM((_NEV + 8, _OU), f32),
            pltpu.VMEM((_T, _B, _OU), f32),
        ],
    )(wkt, p_evt, gl, at, e_flat, Wl, bl2)

    output = out_pre.reshape(_B, _T, _OU)
    final_mem = jnp.broadcast_to(mfin[None], (_B, _N, _BU))
    return output, final_mem


# speedup vs baseline: 1.1494x; 1.0347x over previous
import functools

import jax
import jax.numpy as jnp
import numpy as np
from jax import lax
from jax.experimental import pallas as pl
from jax.experimental.pallas import tpu as pltpu
from jax.experimental.pallas import tpu_sc as plsc

_IU = 256
_BU = 64
_N = 128
_N1 = _N + 1
_NP = 256
_OU = 64
_B = 8
_T = 512

_pallas_call = pl.pallas_call


def _attn_body(x_ref, wq_ref, bq_ref, wk_ref, bk_ref, wa_ref, ba_ref,
               we_ref, be_ref, wk_out, al_out, e_out):
    x = x_ref[0]
    q = jnp.dot(x, wq_ref[...]) + bq_ref[...]
    k = jnp.dot(x, wk_ref[...]) + bk_ref[...]
    am = jnp.dot(x, wa_ref[...]) + ba_ref[...]
    e = jnp.maximum(jnp.dot(x, we_ref[...]) + be_ref[...], 0.0)
    s1 = jax.lax.dot_general(q, k, (((1,), (1,)), ((), ()))) / np.power(
        _N1, 0.5)
    s2 = jnp.dot(s1, am)
    m = jnp.max(s2, axis=0, keepdims=True)
    ez = jnp.exp(s2 - m)
    zs = jnp.sum(ez, axis=0, keepdims=True)
    sm = ez / zs
    col = jax.lax.broadcasted_iota(jnp.int32, (_T, _NP), 1)
    smx = jnp.where(col < _N1, sm, jnp.inf)
    minv = jnp.min(smx, axis=1, keepdims=True)
    idx = jnp.min(jnp.where(smx == minv, col, jnp.int32(1 << 30)), axis=1)
    wk_out[0, 0] = idx
    al_out[0, 0] = sm[_T - 1, :]
    e_out[0] = e


_NEV = _T * _B


_L = 16


def _sc_prev_body(wk_hbm, p_hbm, gl_hbm, wk_v, p_v, gl_v, last_sm):
    cid = lax.axis_index("c")
    sid = lax.axis_index("s")

    @pl.when((cid == 0) & (sid == 0))
    def _():
        pltpu.sync_copy(wk_hbm, wk_v)

        def li(n, _):
            last_sm[n] = -1
            return 0

        lax.fori_loop(0, _N, li, 0)
        lanes = lax.iota(jnp.int32, _L)

        def outer(g, _):
            vec = jnp.zeros((_L,), jnp.int32)
            wkvec = wk_v[pl.ds(g * _L, _L)]
            for r in range(_L):
                i = g * _L + r
                n = wkvec[r]
                valid = n < _N
                sn = jnp.minimum(n, _N - 1)
                pv = jnp.where(valid, last_sm[sn], -1)
                vec = jnp.where(lanes == r, pv, vec)

                @pl.when(valid)
                def _w():
                    last_sm[sn] = i
            p_v[pl.ds(g * _L, _L)] = vec
            return 0

        lax.fori_loop(0, _NEV // _L, outer, 0)

        def gout(g, _):
            vec = jnp.zeros((_L,), jnp.int32)
            for r in range(_L):
                vec = jnp.where(lanes == r, last_sm[g * _L + r], vec)
            gl_v[pl.ds(g * _L, _L)] = vec
            return 0

        lax.fori_loop(0, _N // _L, gout, 0)
        pltpu.sync_copy(p_v, p_hbm)
        pltpu.sync_copy(gl_v, gl_hbm)


@functools.lru_cache(maxsize=1)
def _sc_prev_call():
    mesh = plsc.VectorSubcoreMesh(core_axis_name="c", subcore_axis_name="s")
    return functools.partial(
        pl.kernel,
        mesh=mesh,
        out_type=[
            jax.ShapeDtypeStruct((_NEV,), jnp.int32),
            jax.ShapeDtypeStruct((_N,), jnp.int32),
        ],
        scratch_types=[
            pltpu.VMEM((_NEV,), jnp.int32),
            pltpu.VMEM((_NEV,), jnp.int32),
            pltpu.VMEM((_N,), jnp.int32),
            pltpu.SMEM((_N,), jnp.int32),
        ],
    )(_sc_prev_body)


def _scan_body(wkt_ref, p_ref, gl_ref, atv_ref, e_ref, wl_ref, bl_ref,
               out_ref, mf_ref, d_ref, s3_ref):
    hi = jax.lax.Precision.HIGHEST
    d_ref[pl.ds(_NEV, 1), :] = jnp.zeros((1, _OU), jnp.float32)

    def p13(g, _):
        for u in range(4):
            t = g * 4 + u
            dns = []
            dps = []
            crows = []
            for j in range(_B):
                n = wkt_ref[t, j]
                valid = n < _N
                sn = jnp.minimum(n, _N - 1)
                i = t * _B + j
                e = e_ref[pl.ds(j * _T + t, 1), :]
                wb = wl_ref[pl.ds(sn * _BU, _BU), :]
                dnew = jnp.dot(e, wb, preferred_element_type=jnp.float32,
                               precision=hi)
                d_ref[pl.ds(i, 1), :] = dnew
                dns.append(dnew)
                prev = p_ref[i]
                sp = jnp.where(prev < 0, _NEV, prev)
                dps.append(d_ref[pl.ds(sp, 1), :])
                arow = atv_ref[pl.ds(sn, 1), :]
                crows.append(jnp.where(valid, arow, 0.0))
            delta = (jnp.concatenate(dns, axis=0)
                     - jnp.concatenate(dps, axis=0))
            c = jnp.concatenate(crows, axis=0)
            s = jax.lax.dot_general(c, delta, (((0,), (0,)), ((), ())),
                                    precision=hi)
            s3_ref[pl.ds(t, 1)] = jnp.reshape(s, (1, _B, _OU))
        return 0

    jax.lax.fori_loop(0, _T // 4, p13, 0)

    row = jax.lax.broadcasted_iota(jnp.int32, (_T, _T), 0)
    colt = jax.lax.broadcasted_iota(jnp.int32, (_T, _T), 1)
    ltri = jnp.where(row > colt, 1.0, 0.0).astype(jnp.float32)
    out1 = jax.lax.dot_general(ltri, s3_ref[...], (((1,), (0,)), ((), ())),
                               precision=hi)
    out_ref[...] = out1 + jnp.reshape(bl_ref[...], (1, 1, _OU))
    wl_last = wl_ref[pl.ds(_N * _BU, _BU), :]
    for b in range(_B):
        eb = e_ref[pl.ds(b * _T, _T), :]
        o2 = jnp.dot(eb, wl_last, preferred_element_type=jnp.float32,
                     precision=hi) * atv_ref[pl.ds(_N, 1), pl.ds(b, 1)]
        out_ref[:, pl.ds(b, 1), :] += o2[:, None, :]

    def p5(n, _):
        li = gl_ref[n]
        t = li // _B
        j = li - t * _B
        erow = e_ref[pl.ds(j * _T + jnp.maximum(t, 0), 1), :]
        mf_ref[pl.ds(n, 1), :] = jnp.where(li >= 0, erow, 0.0)
        return 0

    jax.lax.fori_loop(0, _N, p5, 0)


def kernel(inp, Wq, bq, Wk, bk, Wa, ba, We, be, Wl, bl):
    f32 = jnp.float32
    pad = _NP - _N1
    wq_p = jnp.pad(Wq, ((0, 0), (0, pad)))
    wk_p = jnp.pad(Wk, ((0, 0), (0, pad)))
    wa_p = jnp.pad(Wa, ((0, 0), (0, pad)))
    bq_p = jnp.pad(bq, (0, pad)).reshape(1, _NP)
    bk_p = jnp.pad(bk, (0, pad)).reshape(1, _NP)
    ba_p = jnp.pad(ba, (0, pad)).reshape(1, _NP)
    be2 = be.reshape(1, _BU)

    wfull = pl.BlockSpec((_IU, _NP), lambda b: (0, 0))
    bfull = pl.BlockSpec((1, _NP), lambda b: (0, 0))
    weakest, al, e_all = _pallas_call(
        _attn_body,
        grid=(_B,),
        in_specs=[
            pl.BlockSpec((1, _T, _IU), lambda b: (b, 0, 0)),
            wfull, bfull, wfull, bfull, wfull, bfull,
            pl.BlockSpec((_IU, _BU), lambda b: (0, 0)),
            pl.BlockSpec((1, _BU), lambda b: (0, 0)),
        ],
        out_specs=[
            pl.BlockSpec((1, 1, _T), lambda b: (b, 0, 0)),
            pl.BlockSpec((1, 1, _NP), lambda b: (b, 0, 0)),
            pl.BlockSpec((1, _T, _BU), lambda b: (b, 0, 0)),
        ],
        out_shape=[
            jax.ShapeDtypeStruct((_B, 1, _T), jnp.int32),
            jax.ShapeDtypeStruct((_B, 1, _NP), f32),
            jax.ShapeDtypeStruct((_B, _T, _BU), f32),
        ],
    )(inp, wq_p, bq_p, wk_p, bk_p, wa_p, ba_p, We, be2)

    wkt = weakest[:, 0, :].T
    at = al[:, 0, :_N1].T
    e_flat = e_all.reshape(_B * _T, _BU)
    bl2 = bl.reshape(1, _OU)
    p_evt, gl = _sc_prev_call()(wkt.reshape(_NEV))

    out_pre, mfin = _pallas_call(
        _scan_body,
        in_specs=[
            pl.BlockSpec(memory_space=pltpu.SMEM),
            pl.BlockSpec(memory_space=pltpu.SMEM),
            pl.BlockSpec(memory_space=pltpu.SMEM),
            pl.BlockSpec(memory_space=pltpu.VMEM),
            pl.BlockSpec(memory_space=pltpu.VMEM),
            pl.BlockSpec(memory_space=pltpu.VMEM),
            pl.BlockSpec(memory_space=pltpu.VMEM),
        ],
        out_specs=[
            pl.BlockSpec(memory_space=pltpu.VMEM),
            pl.BlockSpec(memory_space=pltpu.VMEM),
        ],
        out_shape=[
            jax.ShapeDtypeStruct((_T, _B, _OU), f32),
            jax.ShapeDtypeStruct((_N, _BU), f32),
        ],
        scratch_shapes=[
            pltpu.VMEM((_NEV + 8, _OU), f32),
            pltpu.VMEM((_T, _B, _OU), f32),
        ],
    )(wkt, p_evt, gl, at, e_flat, Wl, bl2)

    output = out_pre.reshape(_B, _T, _OU)
    final_mem = jnp.broadcast_to(mfin[None], (_B, _N, _BU))
    return output, final_mem


# speedup vs baseline: 1.1903x; 1.0355x over previous
import functools

import jax
import jax.numpy as jnp
import numpy as np
from jax import lax
from jax.experimental import pallas as pl
from jax.experimental.pallas import tpu as pltpu
from jax.experimental.pallas import tpu_sc as plsc

_IU = 256
_BU = 64
_N = 128
_N1 = _N + 1
_NP = 256
_OU = 64
_B = 8
_T = 512

_pallas_call = pl.pallas_call


def _attn_body(x_ref, wq_ref, bq_ref, wk_ref, bk_ref, wa_ref, ba_ref,
               we_ref, be_ref, wk_out, al_out, e_out):
    x = x_ref[0]
    q = jnp.dot(x, wq_ref[...]) + bq_ref[...]
    k = jnp.dot(x, wk_ref[...]) + bk_ref[...]
    am = jnp.dot(x, wa_ref[...]) + ba_ref[...]
    e = jnp.maximum(jnp.dot(x, we_ref[...]) + be_ref[...], 0.0)
    s1 = jax.lax.dot_general(q, k, (((1,), (1,)), ((), ()))) / np.power(
        _N1, 0.5)
    s2 = jnp.dot(s1, am)
    m = jnp.max(s2, axis=0, keepdims=True)
    ez = jnp.exp(s2 - m)
    zs = jnp.sum(ez, axis=0, keepdims=True)
    sm = ez / zs
    col = jax.lax.broadcasted_iota(jnp.int32, (_T, _NP), 1)
    smx = jnp.where(col < _N1, sm, jnp.inf)
    minv = jnp.min(smx, axis=1, keepdims=True)
    idx = jnp.min(jnp.where(smx == minv, col, jnp.int32(1 << 30)), axis=1)
    wk_out[0, 0] = idx
    al_out[0, 0] = sm[_T - 1, :]
    e_out[0] = e


_NEV = _T * _B


_L = 16


def _sc_prev_body(wk_hbm, p_hbm, gl_hbm, wk_v, p_v, gl_v, last_sm):
    cid = lax.axis_index("c")
    sid = lax.axis_index("s")

    @pl.when((cid == 0) & (sid == 0))
    def _():
        pltpu.sync_copy(wk_hbm, wk_v)

        def li(n, _):
            last_sm[n] = -1
            return 0

        lax.fori_loop(0, _N, li, 0)
        lanes = lax.iota(jnp.int32, _L)

        def outer(g, _):
            vec = jnp.zeros((_L,), jnp.int32)
            wkvec = wk_v[pl.ds(g * _L, _L)]
            for r in range(_L):
                i = g * _L + r
                n = wkvec[r]
                valid = n < _N
                sn = jnp.minimum(n, _N - 1)
                pv = jnp.where(valid, last_sm[sn], -1)
                vec = jnp.where(lanes == r, pv, vec)

                @pl.when(valid)
                def _w():
                    last_sm[sn] = i
            p_v[pl.ds(g * _L, _L)] = vec
            return 0

        lax.fori_loop(0, _NEV // _L, outer, 0)

        def gout(g, _):
            vec = jnp.zeros((_L,), jnp.int32)
            for r in range(_L):
                vec = jnp.where(lanes == r, last_sm[g * _L + r], vec)
            gl_v[pl.ds(g * _L, _L)] = vec
            return 0

        lax.fori_loop(0, _N // _L, gout, 0)
        pltpu.sync_copy(p_v, p_hbm)
        pltpu.sync_copy(gl_v, gl_hbm)


@functools.lru_cache(maxsize=1)
def _sc_prev_call():
    mesh = plsc.VectorSubcoreMesh(core_axis_name="c", subcore_axis_name="s")
    return functools.partial(
        pl.kernel,
        mesh=mesh,
        out_type=[
            jax.ShapeDtypeStruct((_NEV,), jnp.int32),
            jax.ShapeDtypeStruct((_N,), jnp.int32),
        ],
        scratch_types=[
            pltpu.VMEM((_NEV,), jnp.int32),
            pltpu.VMEM((_NEV,), jnp.int32),
            pltpu.VMEM((_N,), jnp.int32),
            pltpu.SMEM((_N,), jnp.int32),
        ],
    )(_sc_prev_body)


def _scan_body(wkt_ref, p_ref, gl_ref, atv_ref, e_ref, wl_ref, bl_ref,
               out_ref, mf_ref, d_ref, s3_ref):
    hi = jax.lax.Precision.HIGHEST
    d_ref[pl.ds(_NEV, 1), :] = jnp.zeros((1, _OU), jnp.float32)

    def p13(g, _):
        for u in range(8):
            t = g * 8 + u
            dns = []
            dps = []
            crows = []
            for j in range(_B):
                n = wkt_ref[t, j]
                valid = n < _N
                sn = jnp.minimum(n, _N - 1)
                i = t * _B + j
                e = e_ref[pl.ds(j * _T + t, 1), :]
                wb = wl_ref[pl.ds(sn * _BU, _BU), :]
                dnew = jnp.dot(e, wb, preferred_element_type=jnp.float32,
                               precision=hi)
                d_ref[pl.ds(i, 1), :] = dnew
                dns.append(dnew)
                prev = p_ref[i]
                sp = jnp.where(prev < 0, _NEV, prev)
                dps.append(d_ref[pl.ds(sp, 1), :])
                arow = atv_ref[pl.ds(sn, 1), :]
                crows.append(jnp.where(valid, arow, 0.0))
            delta = (jnp.concatenate(dns, axis=0)
                     - jnp.concatenate(dps, axis=0))
            c = jnp.concatenate(crows, axis=0)
            s = jax.lax.dot_general(c, delta, (((0,), (0,)), ((), ())),
                                    precision=hi)
            s3_ref[pl.ds(t, 1)] = jnp.reshape(s, (1, _B, _OU))
        return 0

    jax.lax.fori_loop(0, _T // 8, p13, 0)

    row = jax.lax.broadcasted_iota(jnp.int32, (_T, _T), 0)
    colt = jax.lax.broadcasted_iota(jnp.int32, (_T, _T), 1)
    ltri = jnp.where(row > colt, 1.0, 0.0).astype(jnp.float32)
    out1 = jax.lax.dot_general(ltri, s3_ref[...], (((1,), (0,)), ((), ())),
                               precision=hi)
    out_ref[...] = out1 + jnp.reshape(bl_ref[...], (1, 1, _OU))
    wl_last = wl_ref[pl.ds(_N * _BU, _BU), :]
    for b in range(_B):
        eb = e_ref[pl.ds(b * _T, _T), :]
        o2 = jnp.dot(eb, wl_last, preferred_element_type=jnp.float32,
                     precision=hi) * atv_ref[pl.ds(_N, 1), pl.ds(b, 1)]
        out_ref[:, pl.ds(b, 1), :] += o2[:, None, :]

    def p5(n, _):
        li = gl_ref[n]
        t = li // _B
        j = li - t * _B
        erow = e_ref[pl.ds(j * _T + jnp.maximum(t, 0), 1), :]
        mf_ref[pl.ds(n, 1), :] = jnp.where(li >= 0, erow, 0.0)
        return 0

    jax.lax.fori_loop(0, _N, p5, 0)


def kernel(inp, Wq, bq, Wk, bk, Wa, ba, We, be, Wl, bl):
    f32 = jnp.float32
    pad = _NP - _N1
    wq_p = jnp.pad(Wq, ((0, 0), (0, pad)))
    wk_p = jnp.pad(Wk, ((0, 0), (0, pad)))
    wa_p = jnp.pad(Wa, ((0, 0), (0, pad)))
    bq_p = jnp.pad(bq, (0, pad)).reshape(1, _NP)
    bk_p = jnp.pad(bk, (0, pad)).reshape(1, _NP)
    ba_p = jnp.pad(ba, (0, pad)).reshape(1, _NP)
    be2 = be.reshape(1, _BU)

    wfull = pl.BlockSpec((_IU, _NP), lambda b: (0, 0))
    bfull = pl.BlockSpec((1, _NP), lambda b: (0, 0))
    weakest, al, e_all = _pallas_call(
        _attn_body,
        grid=(_B,),
        in_specs=[
            pl.BlockSpec((1, _T, _IU), lambda b: (b, 0, 0)),
            wfull, bfull, wfull, bfull, wfull, bfull,
            pl.BlockSpec((_IU, _BU), lambda b: (0, 0)),
            pl.BlockSpec((1, _BU), lambda b: (0, 0)),
        ],
        out_specs=[
            pl.BlockSpec((1, 1, _T), lambda b: (b, 0, 0)),
            pl.BlockSpec((1, 1, _NP), lambda b: (b, 0, 0)),
            pl.BlockSpec((1, _T, _BU), lambda b: (b, 0, 0)),
        ],
        out_shape=[
            jax.ShapeDtypeStruct((_B, 1, _T), jnp.int32),
            jax.ShapeDtypeStruct((_B, 1, _NP), f32),
            jax.ShapeDtypeStruct((_B, _T, _BU), f32),
        ],
    )(inp, wq_p, bq_p, wk_p, bk_p, wa_p, ba_p, We, be2)

    wkt = weakest[:, 0, :].T
    at = al[:, 0, :_N1].T
    e_flat = e_all.reshape(_B * _T, _BU)
    bl2 = bl.reshape(1, _OU)
    p_evt, gl = _sc_prev_call()(wkt.reshape(_NEV))

    out_pre, mfin = _pallas_call(
        _scan_body,
        in_specs=[
            pl.BlockSpec(memory_space=pltpu.SMEM),
            pl.BlockSpec(memory_space=pltpu.SMEM),
            pl.BlockSpec(memory_space=pltpu.SMEM),
            pl.BlockSpec(memory_space=pltpu.VMEM),
            pl.BlockSpec(memory_space=pltpu.VMEM),
            pl.BlockSpec(memory_space=pltpu.VMEM),
            pl.BlockSpec(memory_space=pltpu.VMEM),
        ],
        out_specs=[
            pl.BlockSpec(memory_space=pltpu.VMEM),
            pl.BlockSpec(memory_space=pltpu.VMEM),
        ],
        out_shape=[
            jax.ShapeDtypeStruct((_T, _B, _OU), f32),
            jax.ShapeDtypeStruct((_N, _BU), f32),
        ],
        scratch_shapes=[
            pltpu.VMEM((_NEV + 8, _OU), f32),
            pltpu.VMEM((_T, _B, _OU), f32),
        ],
    )(wkt, p_evt, gl, at, e_flat, Wl, bl2)

    output = out_pre.reshape(_B, _T, _OU)
    final_mem = jnp.broadcast_to(mfin[None], (_B, _N, _BU))
    return output, final_mem


# speedup vs baseline: 1.4754x; 1.2396x over previous
import functools

import jax
import jax.numpy as jnp
import numpy as np
from jax import lax
from jax.experimental import pallas as pl
from jax.experimental.pallas import tpu as pltpu
from jax.experimental.pallas import tpu_sc as plsc

_IU = 256
_BU = 64
_N = 128
_N1 = _N + 1
_NP = 256
_OU = 64
_B = 8
_T = 512

_pallas_call = pl.pallas_call


def _attn_body(x_ref, wq_ref, bq_ref, wk_ref, bk_ref, wa_ref, ba_ref,
               we_ref, be_ref, wk_out, al_out, e_out):
    x = x_ref[0]
    q = jnp.dot(x, wq_ref[...]) + bq_ref[...]
    k = jnp.dot(x, wk_ref[...]) + bk_ref[...]
    am = jnp.dot(x, wa_ref[...]) + ba_ref[...]
    e = jnp.maximum(jnp.dot(x, we_ref[...]) + be_ref[...], 0.0)
    s1 = jax.lax.dot_general(q, k, (((1,), (1,)), ((), ()))) / np.power(
        _N1, 0.5)
    s2 = jnp.dot(s1, am)
    m = jnp.max(s2, axis=0, keepdims=True)
    ez = jnp.exp(s2 - m)
    zs = jnp.sum(ez, axis=0, keepdims=True)
    sm = ez / zs
    col = jax.lax.broadcasted_iota(jnp.int32, (_T, _NP), 1)
    smx = jnp.where(col < _N1, sm, jnp.inf)
    minv = jnp.min(smx, axis=1, keepdims=True)
    idx = jnp.min(jnp.where(smx == minv, col, jnp.int32(1 << 30)), axis=1)
    wk_out[0, 0] = idx
    al_out[0, 0] = sm[_T - 1, :]
    e_out[0] = e


_NEV = _T * _B


_L = 16


def _sc_prev_body(wk_hbm, p_hbm, gl_hbm, wk_v, p_v, gl_v, last_sm):
    cid = lax.axis_index("c")
    sid = lax.axis_index("s")

    @pl.when((cid == 0) & (sid == 0))
    def _():
        pltpu.sync_copy(wk_hbm, wk_v)

        def li(n, _):
            last_sm[n] = -1
            return 0

        lax.fori_loop(0, _N, li, 0)
        lanes = lax.iota(jnp.int32, _L)

        def outer(g, _):
            vec = jnp.zeros((_L,), jnp.int32)
            wkvec = wk_v[pl.ds(g * _L, _L)]
            for r in range(_L):
                i = g * _L + r
                n = wkvec[r]
                valid = n < _N
                sn = jnp.minimum(n, _N - 1)
                pv = jnp.where(valid, last_sm[sn], -1)
                vec = jnp.where(lanes == r, pv, vec)

                @pl.when(valid)
                def _w():
                    last_sm[sn] = i
            p_v[pl.ds(g * _L, _L)] = vec
            return 0

        lax.fori_loop(0, _NEV // _L, outer, 0)

        def gout(g, _):
            vec = jnp.zeros((_L,), jnp.int32)
            for r in range(_L):
                vec = jnp.where(lanes == r, last_sm[g * _L + r], vec)
            gl_v[pl.ds(g * _L, _L)] = vec
            return 0

        lax.fori_loop(0, _N // _L, gout, 0)
        pltpu.sync_copy(p_v, p_hbm)
        pltpu.sync_copy(gl_v, gl_hbm)


@functools.lru_cache(maxsize=1)
def _sc_prev_call():
    mesh = plsc.VectorSubcoreMesh(core_axis_name="c", subcore_axis_name="s")
    return functools.partial(
        pl.kernel,
        mesh=mesh,
        out_type=[
            jax.ShapeDtypeStruct((_NEV,), jnp.int32),
            jax.ShapeDtypeStruct((_N,), jnp.int32),
        ],
        scratch_types=[
            pltpu.VMEM((_NEV,), jnp.int32),
            pltpu.VMEM((_NEV,), jnp.int32),
            pltpu.VMEM((_N,), jnp.int32),
            pltpu.SMEM((_N,), jnp.int32),
        ],
    )(_sc_prev_body)


def _scan_body(wkt_ref, p_ref, gl_ref, atv_ref, e_ref, wl_ref, bl_ref,
               out_ref, mf_ref, d_ref, s3_ref):
    hi = jax.lax.Precision.HIGHEST
    d_ref[pl.ds(_NEV, 1), :] = jnp.zeros((1, _OU), jnp.float32)

    def p1(g, _):
        for u in range(8):
            t = g * 8 + u
            for j in range(_B):
                n = wkt_ref[t, j]
                sn = jnp.minimum(n, _N - 1)
                e = e_ref[pl.ds(j * _T + t, 1), :]
                wb = wl_ref[pl.ds(sn * _BU, _BU), :]
                d_ref[pl.ds(t * _B + j, 1), :] = jnp.dot(
                    e, wb, preferred_element_type=jnp.float32, precision=hi)
        return 0

    jax.lax.fori_loop(0, _T // 8, p1, 0)

    def p3(g, _):
        for u in range(8):
            t = g * 8 + u
            dps = []
            crows = []
            for j in range(_B):
                n = wkt_ref[t, j]
                valid = n < _N
                sn = jnp.minimum(n, _N - 1)
                i = t * _B + j
                prev = p_ref[i]
                sp = jnp.where(prev < 0, _NEV, prev)
                dps.append(d_ref[pl.ds(sp, 1), :])
                arow = atv_ref[pl.ds(sn, 1), :]
                crows.append(jnp.where(valid, arow, 0.0))
            dn = d_ref[pl.ds(t * _B, _B), :]
            delta = dn - jnp.concatenate(dps, axis=0)
            c = jnp.concatenate(crows, axis=0)
            s = jax.lax.dot_general(c, delta, (((0,), (0,)), ((), ())),
                                    precision=hi)
            s3_ref[pl.ds(t, 1)] = jnp.reshape(s, (1, _B, _OU))
        return 0

    jax.lax.fori_loop(0, _T // 8, p3, 0)

    row = jax.lax.broadcasted_iota(jnp.int32, (_T, _T), 0)
    colt = jax.lax.broadcasted_iota(jnp.int32, (_T, _T), 1)
    ltri = jnp.where(row > colt, 1.0, 0.0).astype(jnp.float32)
    out1 = jax.lax.dot_general(ltri, s3_ref[...], (((1,), (0,)), ((), ())),
                               precision=hi)
    out_ref[...] = out1 + jnp.reshape(bl_ref[...], (1, 1, _OU))
    wl_last = wl_ref[pl.ds(_N * _BU, _BU), :]
    for b in range(_B):
        eb = e_ref[pl.ds(b * _T, _T), :]
        o2 = jnp.dot(eb, wl_last, preferred_element_type=jnp.float32,
                     precision=hi) * atv_ref[pl.ds(_N, 1), pl.ds(b, 1)]
        out_ref[:, pl.ds(b, 1), :] += o2[:, None, :]

    def p5(n, _):
        li = gl_ref[n]
        t = li // _B
        j = li - t * _B
        erow = e_ref[pl.ds(j * _T + jnp.maximum(t, 0), 1), :]
        mf_ref[pl.ds(n, 1), :] = jnp.where(li >= 0, erow, 0.0)
        return 0

    jax.lax.fori_loop(0, _N, p5, 0)


def kernel(inp, Wq, bq, Wk, bk, Wa, ba, We, be, Wl, bl):
    f32 = jnp.float32
    pad = _NP - _N1
    wq_p = jnp.pad(Wq, ((0, 0), (0, pad)))
    wk_p = jnp.pad(Wk, ((0, 0), (0, pad)))
    wa_p = jnp.pad(Wa, ((0, 0), (0, pad)))
    bq_p = jnp.pad(bq, (0, pad)).reshape(1, _NP)
    bk_p = jnp.pad(bk, (0, pad)).reshape(1, _NP)
    ba_p = jnp.pad(ba, (0, pad)).reshape(1, _NP)
    be2 = be.reshape(1, _BU)

    wfull = pl.BlockSpec((_IU, _NP), lambda b: (0, 0))
    bfull = pl.BlockSpec((1, _NP), lambda b: (0, 0))
    weakest, al, e_all = _pallas_call(
        _attn_body,
        grid=(_B,),
        in_specs=[
            pl.BlockSpec((1, _T, _IU), lambda b: (b, 0, 0)),
            wfull, bfull, wfull, bfull, wfull, bfull,
            pl.BlockSpec((_IU, _BU), lambda b: (0, 0)),
            pl.BlockSpec((1, _BU), lambda b: (0, 0)),
        ],
        out_specs=[
            pl.BlockSpec((1, 1, _T), lambda b: (b, 0, 0)),
            pl.BlockSpec((1, 1, _NP), lambda b: (b, 0, 0)),
            pl.BlockSpec((1, _T, _BU), lambda b: (b, 0, 0)),
        ],
        out_shape=[
            jax.ShapeDtypeStruct((_B, 1, _T), jnp.int32),
            jax.ShapeDtypeStruct((_B, 1, _NP), f32),
            jax.ShapeDtypeStruct((_B, _T, _BU), f32),
        ],
    )(inp, wq_p, bq_p, wk_p, bk_p, wa_p, ba_p, We, be2)

    wkt = weakest[:, 0, :].T
    at = al[:, 0, :_N1].T
    e_flat = e_all.reshape(_B * _T, _BU)
    bl2 = bl.reshape(1, _OU)
    p_evt, gl = _sc_prev_call()(wkt.reshape(_NEV))

    out_pre, mfin = _pallas_call(
        _scan_body,
        in_specs=[
            pl.BlockSpec(memory_space=pltpu.SMEM),
            pl.BlockSpec(memory_space=pltpu.SMEM),
            pl.BlockSpec(memory_space=pltpu.SMEM),
            pl.BlockSpec(memory_space=pltpu.VMEM),
            pl.BlockSpec(memory_space=pltpu.VMEM),
            pl.BlockSpec(memory_space=pltpu.VMEM),
            pl.BlockSpec(memory_space=pltpu.VMEM),
        ],
        out_specs=[
            pl.BlockSpec(memory_space=pltpu.VMEM),
            pl.BlockSpec(memory_space=pltpu.VMEM),
        ],
        out_shape=[
            jax.ShapeDtypeStruct((_T, _B, _OU), f32),
            jax.ShapeDtypeStruct((_N, _BU), f32),
        ],
        scratch_shapes=[
            pltpu.VMEM((_NEV + 8, _OU), f32),
            pltpu.VMEM((_T, _B, _OU), f32),
        ],
    )(wkt, p_evt, gl, at, e_flat, Wl, bl2)

    output = out_pre.reshape(_B, _T, _OU)
    final_mem = jnp.broadcast_to(mfin[None], (_B, _N, _BU))
    return output, final_mem


# speedup vs baseline: 1.5337x; 1.0395x over previous
import functools

import jax
import jax.numpy as jnp
import numpy as np
from jax import lax
from jax.experimental import pallas as pl
from jax.experimental.pallas import tpu as pltpu
from jax.experimental.pallas import tpu_sc as plsc

_IU = 256
_BU = 64
_N = 128
_N1 = _N + 1
_NP = 256
_OU = 64
_B = 8
_T = 512

_pallas_call = pl.pallas_call


def _attn_body(x_ref, wq_ref, bq_ref, wk_ref, bk_ref, wa_ref, ba_ref,
               we_ref, be_ref, wk_out, al_out, e_out):
    x = x_ref[0]
    q = jnp.dot(x, wq_ref[...]) + bq_ref[...]
    k = jnp.dot(x, wk_ref[...]) + bk_ref[...]
    am = jnp.dot(x, wa_ref[...]) + ba_ref[...]
    e = jnp.maximum(jnp.dot(x, we_ref[...]) + be_ref[...], 0.0)
    s1 = jax.lax.dot_general(q, k, (((1,), (1,)), ((), ()))) / np.power(
        _N1, 0.5)
    s2 = jnp.dot(s1, am)
    m = jnp.max(s2, axis=0, keepdims=True)
    ez = jnp.exp(s2 - m)
    zs = jnp.sum(ez, axis=0, keepdims=True)
    sm = ez / zs
    col = jax.lax.broadcasted_iota(jnp.int32, (_T, _NP), 1)
    smx = jnp.where(col < _N1, sm, jnp.inf)
    minv = jnp.min(smx, axis=1, keepdims=True)
    idx = jnp.min(jnp.where(smx == minv, col, jnp.int32(1 << 30)), axis=1)
    wk_out[0, 0] = idx
    al_out[0, 0] = sm[_T - 1, :]
    e_out[0] = e


_NEV = _T * _B


_L = 16


def _sc_prev_body(wk_hbm, p_hbm, gl_hbm, wk_v, p_v, gl_v, last_sm):
    cid = lax.axis_index("c")
    sid = lax.axis_index("s")

    @pl.when((cid == 0) & (sid == 0))
    def _():
        pltpu.sync_copy(wk_hbm, wk_v)

        def li(n, _):
            last_sm[n] = -1
            return 0

        lax.fori_loop(0, _N, li, 0)
        lanes = lax.iota(jnp.int32, _L)

        def outer(g, _):
            vec = jnp.zeros((_L,), jnp.int32)
            wkvec = wk_v[pl.ds(g * _L, _L)]
            for r in range(_L):
                i = g * _L + r
                n = wkvec[r]
                valid = n < _N
                sn = jnp.minimum(n, _N - 1)
                pv = jnp.where(valid, last_sm[sn], -1)
                vec = jnp.where(lanes == r, pv, vec)

                @pl.when(valid)
                def _w():
                    last_sm[sn] = i
            p_v[pl.ds(g * _L, _L)] = vec
            return 0

        lax.fori_loop(0, _NEV // _L, outer, 0)

        def gout(g, _):
            vec = jnp.zeros((_L,), jnp.int32)
            for r in range(_L):
                vec = jnp.where(lanes == r, last_sm[g * _L + r], vec)
            gl_v[pl.ds(g * _L, _L)] = vec
            return 0

        lax.fori_loop(0, _N // _L, gout, 0)
        pltpu.sync_copy(p_v, p_hbm)
        pltpu.sync_copy(gl_v, gl_hbm)


@functools.lru_cache(maxsize=1)
def _sc_prev_call():
    mesh = plsc.VectorSubcoreMesh(core_axis_name="c", subcore_axis_name="s")
    return functools.partial(
        pl.kernel,
        mesh=mesh,
        out_type=[
            jax.ShapeDtypeStruct((_NEV,), jnp.int32),
            jax.ShapeDtypeStruct((_N,), jnp.int32),
        ],
        scratch_types=[
            pltpu.VMEM((_NEV,), jnp.int32),
            pltpu.VMEM((_NEV,), jnp.int32),
            pltpu.VMEM((_N,), jnp.int32),
            pltpu.SMEM((_N,), jnp.int32),
        ],
    )(_sc_prev_body)


def _scan_body(wkt_ref, p_ref, gl_ref, atv_ref, e_ref, wl_ref, bl_ref,
               out_ref, mf_ref, d_ref, s3_ref):
    hi = jax.lax.Precision.HIGHEST
    d_ref[pl.ds(_NEV, 1), :] = jnp.zeros((1, _OU), jnp.float32)

    def p1(g, _):
        for u in range(16):
            t = g * 16 + u
            for j in range(_B):
                n = wkt_ref[t, j]
                sn = jnp.minimum(n, _N - 1)
                e = e_ref[pl.ds(j * _T + t, 1), :]
                wb = wl_ref[pl.ds(sn * _BU, _BU), :]
                d_ref[pl.ds(t * _B + j, 1), :] = jnp.dot(
                    e, wb, preferred_element_type=jnp.float32, precision=hi)
        return 0

    jax.lax.fori_loop(0, _T // 16, p1, 0)

    def p3(g, _):
        for u in range(16):
            t = g * 16 + u
            dps = []
            crows = []
            for j in range(_B):
                n = wkt_ref[t, j]
                valid = n < _N
                sn = jnp.minimum(n, _N - 1)
                i = t * _B + j
                prev = p_ref[i]
                sp = jnp.where(prev < 0, _NEV, prev)
                dps.append(d_ref[pl.ds(sp, 1), :])
                arow = atv_ref[pl.ds(sn, 1), :]
                crows.append(jnp.where(valid, arow, 0.0))
            dn = d_ref[pl.ds(t * _B, _B), :]
            delta = dn - jnp.concatenate(dps, axis=0)
            c = jnp.concatenate(crows, axis=0)
            s = jax.lax.dot_general(c, delta, (((0,), (0,)), ((), ())),
                                    precision=hi)
            s3_ref[pl.ds(t, 1)] = jnp.reshape(s, (1, _B, _OU))
        return 0

    jax.lax.fori_loop(0, _T // 16, p3, 0)

    row = jax.lax.broadcasted_iota(jnp.int32, (_T, _T), 0)
    colt = jax.lax.broadcasted_iota(jnp.int32, (_T, _T), 1)
    ltri = jnp.where(row > colt, 1.0, 0.0).astype(jnp.float32)
    out1 = jax.lax.dot_general(ltri, s3_ref[...], (((1,), (0,)), ((), ())),
                               precision=hi)
    out_ref[...] = out1 + jnp.reshape(bl_ref[...], (1, 1, _OU))
    wl_last = wl_ref[pl.ds(_N * _BU, _BU), :]
    for b in range(_B):
        eb = e_ref[pl.ds(b * _T, _T), :]
        o2 = jnp.dot(eb, wl_last, preferred_element_type=jnp.float32,
                     precision=hi) * atv_ref[pl.ds(_N, 1), pl.ds(b, 1)]
        out_ref[:, pl.ds(b, 1), :] += o2[:, None, :]

    def p5(n, _):
        li = gl_ref[n]
        t = li // _B
        j = li - t * _B
        erow = e_ref[pl.ds(j * _T + jnp.maximum(t, 0), 1), :]
        mf_ref[pl.ds(n, 1), :] = jnp.where(li >= 0, erow, 0.0)
        return 0

    jax.lax.fori_loop(0, _N, p5, 0)


def kernel(inp, Wq, bq, Wk, bk, Wa, ba, We, be, Wl, bl):
    f32 = jnp.float32
    pad = _NP - _N1
    wq_p = jnp.pad(Wq, ((0, 0), (0, pad)))
    wk_p = jnp.pad(Wk, ((0, 0), (0, pad)))
    wa_p = jnp.pad(Wa, ((0, 0), (0, pad)))
    bq_p = jnp.pad(bq, (0, pad)).reshape(1, _NP)
    bk_p = jnp.pad(bk, (0, pad)).reshape(1, _NP)
    ba_p = jnp.pad(ba, (0, pad)).reshape(1, _NP)
    be2 = be.reshape(1, _BU)

    wfull = pl.BlockSpec((_IU, _NP), lambda b: (0, 0))
    bfull = pl.BlockSpec((1, _NP), lambda b: (0, 0))
    weakest, al, e_all = _pallas_call(
        _attn_body,
        grid=(_B,),
        in_specs=[
            pl.BlockSpec((1, _T, _IU), lambda b: (b, 0, 0)),
            wfull, bfull, wfull, bfull, wfull, bfull,
            pl.BlockSpec((_IU, _BU), lambda b: (0, 0)),
            pl.BlockSpec((1, _BU), lambda b: (0, 0)),
        ],
        out_specs=[
            pl.BlockSpec((1, 1, _T), lambda b: (b, 0, 0)),
            pl.BlockSpec((1, 1, _NP), lambda b: (b, 0, 0)),
            pl.BlockSpec((1, _T, _BU), lambda b: (b, 0, 0)),
        ],
        out_shape=[
            jax.ShapeDtypeStruct((_B, 1, _T), jnp.int32),
            jax.ShapeDtypeStruct((_B, 1, _NP), f32),
            jax.ShapeDtypeStruct((_B, _T, _BU), f32),
        ],
    )(inp, wq_p, bq_p, wk_p, bk_p, wa_p, ba_p, We, be2)

    wkt = weakest[:, 0, :].T
    at = al[:, 0, :_N1].T
    e_flat = e_all.reshape(_B * _T, _BU)
    bl2 = bl.reshape(1, _OU)
    p_evt, gl = _sc_prev_call()(wkt.reshape(_NEV))

    out_pre, mfin = _pallas_call(
        _scan_body,
        in_specs=[
            pl.BlockSpec(memory_space=pltpu.SMEM),
            pl.BlockSpec(memory_space=pltpu.SMEM),
            pl.BlockSpec(memory_space=pltpu.SMEM),
            pl.BlockSpec(memory_space=pltpu.VMEM),
            pl.BlockSpec(memory_space=pltpu.VMEM),
            pl.BlockSpec(memory_space=pltpu.VMEM),
            pl.BlockSpec(memory_space=pltpu.VMEM),
        ],
        out_specs=[
            pl.BlockSpec(memory_space=pltpu.VMEM),
            pl.BlockSpec(memory_space=pltpu.VMEM),
        ],
        out_shape=[
            jax.ShapeDtypeStruct((_T, _B, _OU), f32),
            jax.ShapeDtypeStruct((_N, _BU), f32),
        ],
        scratch_shapes=[
            pltpu.VMEM((_NEV + 8, _OU), f32),
            pltpu.VMEM((_T, _B, _OU), f32),
        ],
    )(wkt, p_evt, gl, at, e_flat, Wl, bl2)

    output = out_pre.reshape(_B, _T, _OU)
    final_mem = jnp.broadcast_to(mfin[None], (_B, _N, _BU))
    return output, final_mem
